# baseline (speedup 1.0000x reference)
import functools

import jax
import jax.numpy as jnp
from jax import lax
from jax.experimental import pallas as pl
from jax.experimental.pallas import tpu as pltpu
from jax.experimental.pallas import tpu_sc as plsc


def kernel(x, table):
    B, S, D = x.shape
    info = plsc.get_sparse_core_info()
    NC, NS, L = info.num_cores, info.num_subcores, info.num_lanes
    NW = NC * NS
    rows_per_w = S // NW
    CS = 32
    n_chunks = rows_per_w // CS
    vecs = CS * D // L

    mesh = plsc.VectorSubcoreMesh(core_axis_name="c", subcore_axis_name="s")

    @functools.partial(
        pl.kernel,
        out_type=jax.ShapeDtypeStruct((B, S * D), jnp.float32),
        mesh=mesh,
        scratch_types=[
            pltpu.VMEM((CS * D,), jnp.float32),
            pltpu.VMEM((CS * D,), jnp.float32),
        ],
    )
    def sc_add(x_hbm, t_hbm, out_hbm, tbuf, xbuf):
        wid = lax.axis_index("s") * NC + lax.axis_index("c")
        base = wid * rows_per_w
        for c in range(n_chunks):
            s0 = (base + c * CS) * D
            pltpu.sync_copy(t_hbm.at[pl.ds(s0, CS * D)], tbuf)
            for b in range(B):
                pltpu.sync_copy(x_hbm.at[b, pl.ds(s0, CS * D)], xbuf)

                @plsc.parallel_loop(0, vecs, unroll=8)
                def _(i):
                    off = i * L
                    plsc.addupdate(xbuf.at[pl.ds(off, L)], tbuf[pl.ds(off, L)])

                pltpu.sync_copy(xbuf, out_hbm.at[b, pl.ds(s0, CS * D)])

    x_flat = x.reshape(B, S * D)
    t_flat = table[:S].reshape(S * D)
    out = sc_add(x_flat, t_flat)
    return out.reshape(B, S, D)


# speedup vs baseline: 1.2030x; 1.2030x over previous
import functools

import jax
import jax.numpy as jnp
from jax import lax
from jax.experimental import pallas as pl
from jax.experimental.pallas import tpu as pltpu
from jax.experimental.pallas import tpu_sc as plsc

_NXB = 3


def kernel(x, table):
    B, S, D = x.shape
    info = plsc.get_sparse_core_info()
    NC, NS, L = info.num_cores, info.num_subcores, info.num_lanes
    NW = NC * NS
    rows_per_w = S // NW
    CS = 16
    n_chunks = rows_per_w // CS
    vecs = CS * D // L
    nsteps = n_chunks * B

    mesh = plsc.VectorSubcoreMesh(core_axis_name="c", subcore_axis_name="s")

    @functools.partial(
        pl.kernel,
        out_type=jax.ShapeDtypeStruct((B, S * D), jnp.float32),
        mesh=mesh,
        scratch_types=[
            [pltpu.VMEM((CS * D,), jnp.float32) for _ in range(2)],
            [pltpu.VMEM((CS * D,), jnp.float32) for _ in range(_NXB)],
            [pltpu.SemaphoreType.DMA for _ in range(2)],
            [pltpu.SemaphoreType.DMA for _ in range(_NXB)],
            [pltpu.SemaphoreType.DMA for _ in range(_NXB)],
        ],
    )
    def sc_add(x_hbm, t_hbm, out_hbm, tbufs, xbufs, tsems, xsems, osems):
        wid = lax.axis_index("s") * NC + lax.axis_index("c")
        base = wid * rows_per_w

        def t_load(c):
            s0 = (base + c * CS) * D
            return pltpu.async_copy(
                t_hbm.at[pl.ds(s0, CS * D)], tbufs[c % 2], tsems[c % 2])

        def x_load(step):
            c, b = divmod(step, B)
            s0 = (base + c * CS) * D
            return pltpu.async_copy(
                x_hbm.at[b, pl.ds(s0, CS * D)], xbufs[step % _NXB],
                xsems[step % _NXB])

        def out_store(step):
            c, b = divmod(step, B)
            s0 = (base + c * CS) * D
            return pltpu.async_copy(
                xbufs[step % _NXB], out_hbm.at[b, pl.ds(s0, CS * D)],
                osems[step % _NXB])

        PF = _NXB - 1
        t_pend = t_load(0)
        x_pend = [x_load(s) for s in range(min(PF, nsteps))]
        o_pend = {}

        for s in range(nsteps):
            c, b = divmod(s, B)
            if b == 0 and c + 1 < n_chunks:
                t_next = t_load(c + 1)
            if b == 0:
                t_pend.wait()
            x_pend[s % PF].wait()

            tbuf = tbufs[c % 2]
            xbuf = xbufs[s % _NXB]

            @plsc.parallel_loop(0, vecs, unroll=8)
            def _(i):
                off = i * L
                plsc.addupdate(xbuf.at[pl.ds(off, L)], tbuf[pl.ds(off, L)])

            o_pend[s] = out_store(s)
            if b == B - 1 and c + 1 < n_chunks:
                t_pend = t_next
            nxt = s + PF
            if nxt < nsteps:
                prev = nxt - _NXB
                if prev in o_pend:
                    o_pend.pop(prev).wait()
                x_pend[nxt % PF] = x_load(nxt)

        for s in sorted(o_pend):
            o_pend.pop(s).wait()

    x_flat = x.reshape(B, S * D)
    t_flat = table[:S].reshape(S * D)
    out = sc_add(x_flat, t_flat)
    return out.reshape(B, S, D)


# speedup vs baseline: 3.1446x; 2.6139x over previous
import functools

import jax
import jax.numpy as jnp
from jax import lax
from jax.experimental import pallas as pl
from jax.experimental.pallas import tpu as pltpu
from jax.experimental.pallas import tpu_sc as plsc

_NXB = 3


def kernel(x, table):
    B, S, D = x.shape
    info = plsc.get_sparse_core_info()
    NC, NS, L = info.num_cores, info.num_subcores, info.num_lanes
    NW = NC * NS
    rows_per_w = S // NW
    CS = 16
    n_chunks = rows_per_w // CS
    vecs = CS * D // L
    nsteps = n_chunks * B

    mesh = plsc.VectorSubcoreMesh(core_axis_name="c", subcore_axis_name="s")

    @functools.partial(
        pl.kernel,
        out_type=jax.ShapeDtypeStruct((B * S * D,), jnp.float32),
        mesh=mesh,
        scratch_types=[
            [pltpu.VMEM((CS * D,), jnp.float32) for _ in range(2)],
            [pltpu.VMEM((CS * D,), jnp.float32) for _ in range(_NXB)],
            [pltpu.SemaphoreType.DMA for _ in range(2)],
            [pltpu.SemaphoreType.DMA for _ in range(_NXB)],
            [pltpu.SemaphoreType.DMA for _ in range(_NXB)],
        ],
    )
    def sc_add(x_hbm, t_hbm, out_hbm, tbufs, xbufs, tsems, xsems, osems):
        wid = lax.axis_index("s") * NC + lax.axis_index("c")
        base = wid * rows_per_w

        def t_load(c):
            s0 = (base + c * CS) * D
            return pltpu.async_copy(
                t_hbm.at[pl.ds(s0, CS * D)], tbufs[c % 2], tsems[c % 2])

        def x_load(step):
            c, b = divmod(step, B)
            s0 = (base + c * CS) * D
            return pltpu.async_copy(
                x_hbm.at[pl.ds(b * S * D + s0, CS * D)], xbufs[step % _NXB],
                xsems[step % _NXB])

        def out_store(step):
            c, b = divmod(step, B)
            s0 = (base + c * CS) * D
            return pltpu.async_copy(
                xbufs[step % _NXB], out_hbm.at[pl.ds(b * S * D + s0, CS * D)],
                osems[step % _NXB])

        PF = _NXB - 1
        t_pend = t_load(0)
        x_pend = [x_load(s) for s in range(min(PF, nsteps))]
        o_pend = {}

        for s in range(nsteps):
            c, b = divmod(s, B)
            if b == 0 and c + 1 < n_chunks:
                t_next = t_load(c + 1)
            if b == 0:
                t_pend.wait()
            x_pend[s % PF].wait()

            tbuf = tbufs[c % 2]
            xbuf = xbufs[s % _NXB]

            @plsc.parallel_loop(0, vecs, unroll=8)
            def _(i):
                off = i * L
                plsc.addupdate(xbuf.at[pl.ds(off, L)], tbuf[pl.ds(off, L)])

            o_pend[s] = out_store(s)
            if b == B - 1 and c + 1 < n_chunks:
                t_pend = t_next
            nxt = s + PF
            if nxt < nsteps:
                prev = nxt - _NXB
                if prev in o_pend:
                    o_pend.pop(prev).wait()
                x_pend[nxt % PF] = x_load(nxt)

        for s in sorted(o_pend):
            o_pend.pop(s).wait()

    x_flat = x.reshape(B * S * D)
    t_flat = table[:S].reshape(S * D)
    out = sc_add(x_flat, t_flat)
    return out.reshape(B, S, D)


# speedup vs baseline: 3.1460x; 1.0005x over previous
import functools

import jax
import jax.numpy as jnp
from jax import lax
from jax.experimental import pallas as pl
from jax.experimental.pallas import tpu as pltpu
from jax.experimental.pallas import tpu_sc as plsc

_NXB = 3


def kernel(x, table):
    B, S, D = x.shape
    info = plsc.get_sparse_core_info()
    NC, NS, L = info.num_cores, info.num_subcores, info.num_lanes
    NW = NC * NS
    rows_per_w = S // NW
    CS = 16
    n_chunks = rows_per_w // CS
    vpr = D // L
    vecs = CS * vpr
    nsteps = n_chunks * B

    mesh = plsc.VectorSubcoreMesh(core_axis_name="c", subcore_axis_name="s")

    @functools.partial(
        pl.kernel,
        out_type=jax.ShapeDtypeStruct((B, S, D), jnp.float32),
        mesh=mesh,
        compiler_params=pltpu.CompilerParams(use_tc_tiling_on_sc=True),
        scratch_types=[
            [pltpu.VMEM((CS, D), jnp.float32) for _ in range(2)],
            [pltpu.VMEM((CS, D), jnp.float32) for _ in range(_NXB)],
            [pltpu.SemaphoreType.DMA for _ in range(2)],
            [pltpu.SemaphoreType.DMA for _ in range(_NXB)],
            [pltpu.SemaphoreType.DMA for _ in range(_NXB)],
        ],
    )
    def sc_add(x_hbm, t_hbm, out_hbm, tbufs, xbufs, tsems, xsems, osems):
        wid = lax.axis_index("s") * NC + lax.axis_index("c")
        base = wid * rows_per_w

        def t_load(c):
            r0 = base + c * CS
            return pltpu.async_copy(
                t_hbm.at[pl.ds(r0, CS), :], tbufs[c % 2], tsems[c % 2])

        def x_load(step):
            c, b = divmod(step, B)
            r0 = base + c * CS
            return pltpu.async_copy(
                x_hbm.at[b, pl.ds(r0, CS), :], xbufs[step % _NXB],
                xsems[step % _NXB])

        def out_store(step):
            c, b = divmod(step, B)
            r0 = base + c * CS
            return pltpu.async_copy(
                xbufs[step % _NXB], out_hbm.at[b, pl.ds(r0, CS), :],
                osems[step % _NXB])

        PF = _NXB - 1
        t_pend = t_load(0)
        x_pend = [x_load(s) for s in range(min(PF, nsteps))]
        o_pend = {}

        for s in range(nsteps):
            c, b = divmod(s, B)
            if b == 0 and c + 1 < n_chunks:
                t_next = t_load(c + 1)
            if b == 0:
                t_pend.wait()
            x_pend[s % PF].wait()

            tbuf = tbufs[c % 2]
            xbuf = xbufs[s % _NXB]

            @plsc.parallel_loop(0, vecs, unroll=8)
            def _(i):
                r = i // vpr
                off = (i % vpr) * L
                plsc.addupdate(xbuf.at[r, pl.ds(off, L)],
                               tbuf[r, pl.ds(off, L)])

            o_pend[s] = out_store(s)
            if b == B - 1 and c + 1 < n_chunks:
                t_pend = t_next
            nxt = s + PF
            if nxt < nsteps:
                prev = nxt - _NXB
                if prev in o_pend:
                    o_pend.pop(prev).wait()
                x_pend[nxt % PF] = x_load(nxt)

        for s in sorted(o_pend):
            o_pend.pop(s).wait()

    return sc_add(x, table)
